# Initial kernel scaffold; baseline (speedup 1.0000x reference)
import jax
import jax.numpy as jnp
from jax.experimental import pallas as pl


def kernel(upper_tri_values):
    raise NotImplementedError("write your pallas kernel here")


# speedup vs baseline: 198.8226x; 198.8226x over previous
import functools

import jax
import jax.numpy as jnp
from jax import lax
from jax.experimental import pallas as pl
from jax.experimental.pallas import tpu as pltpu
from jax.experimental.pallas import tpu_sc as plsc

N = 8192
NN = N * N
NW = 32
ROWS_PER_W = N // NW
CHUNK = 2048
LANES = 16


def _tri(i):
    return i * N - (i * (i - 1)) // 2


def _sc_body(in_hbm, out_hbm, rowbuf, runbuf):
    wid = lax.axis_index("s") * 2 + lax.axis_index("c")

    def per_row(it, carry):
        r = wid + NW * it
        pltpu.sync_copy(in_hbm.at[pl.ds(r * N, N)], rowbuf.at[pl.ds(0, N)])

        k0 = (r * (r + 1)) // 2

        def bs(_, lohi):
            lo, hi = lohi
            mid = (lo + hi) // 2
            ge = _tri(mid) <= k0
            return jnp.where(ge, mid, lo), jnp.where(ge, hi, mid)

        def bs1(_, lohi):
            lo, hi = lohi
            mid = (lo + hi) // 2
            ge = _tri(mid) <= k0 + r
            return jnp.where(ge, mid, lo), jnp.where(ge, hi, mid)

        i0, _ = lax.fori_loop(0, 13, bs, (jnp.int32(0), jnp.int32(N)))
        j0 = i0 + (k0 - _tri(i0))
        i1, _ = lax.fori_loop(0, 13, bs1, (jnp.int32(0), jnp.int32(N)))
        nruns = i1 - i0 + 1

        def run_body(_, state):
            i, j, d = state
            ln = jnp.minimum(N - j, (r + 1) - d)
            s = i * N + j
            a = jnp.minimum((s // 8) * 8, NN - CHUNK)
            h = s - a
            ct = (h + ln + (CHUNK - 1)) // CHUNK

            def chunk(q, c2):
                pltpu.sync_copy(in_hbm.at[pl.ds(a + q * CHUNK, CHUNK)],
                                runbuf.at[pl.ds(q * CHUNK, CHUNK)])
                return c2

            lax.fori_loop(0, ct, chunk, 0)

            nt = (ln + (LANES - 1)) // LANES
            lanes = lax.iota(jnp.int32, LANES)

            def merge(t, c2):
                t16 = t * LANES
                vals = runbuf[pl.ds(h + t16, LANES)]
                old = rowbuf[pl.ds(d + t16, LANES)]
                keep = lanes < (ln - t16)
                rowbuf[pl.ds(d + t16, LANES)] = jnp.where(keep, vals, old)
                return c2

            lax.fori_loop(0, nt, merge, 0)
            return i + 1, i + 1, d + ln

        lax.fori_loop(0, nruns, run_body, (i0, j0, jnp.int32(0)))

        pltpu.sync_copy(rowbuf.at[pl.ds(0, N)], out_hbm.at[pl.ds(r * N, N)])
        return carry

    lax.fori_loop(0, ROWS_PER_W, per_row, jnp.int32(0))


@functools.partial(jax.jit, donate_argnums=())
def _sc_build(x_flat):
    mesh = plsc.VectorSubcoreMesh(core_axis_name="c", subcore_axis_name="s")
    f = pl.kernel(
        _sc_body,
        mesh=mesh,
        out_type=jax.ShapeDtypeStruct((NN,), jnp.float32),
        scratch_types=[
            pltpu.VMEM((N + 32,), jnp.float32),
            pltpu.VMEM((N + CHUNK + 32,), jnp.float32),
        ],
    )
    return f(x_flat)


def kernel(upper_tri_values):
    x = upper_tri_values.reshape(-1)
    return _sc_build(x).reshape(N, N)
